# baseline (speedup 1.0000x reference)
import numpy as np
import jax
import jax.numpy as jnp
from jax.experimental import pallas as pl

_C, _T, _H, _W = 3, 32, 256, 256
_ALPHA = 4
_NSLOW = _T // _ALPHA
_IDX = np.linspace(0.0, _T - 1, _NSLOW).astype(np.int32)
_TB = 16
_NQ = _T // _TB
_SPB = _NSLOW // _NQ
for _q in range(_NQ):
    for _j in range(_SPB):
        assert _TB * _q <= _IDX[_SPB * _q + _j] < _TB * (_q + 1)


def _body(in_ref, slow_ref, fast_ref):
    q = pl.program_id(0)
    fast_ref[...] = in_ref[...]
    for j in range(_SPB):
        i = _SPB * q + j
        off = (31 * i) // 7 - _TB * q
        slow_ref[:, pl.ds(j, 1)] = in_ref[:, pl.ds(off, 1)]


def kernel(frames):
    slow, fast = pl.pallas_call(
        _body,
        grid=(_NQ,),
        in_specs=[pl.BlockSpec((_C, _TB, _H, _W), lambda q: (0, q, 0, 0))],
        out_specs=[
            pl.BlockSpec((_C, _SPB, _H, _W), lambda q: (0, q, 0, 0)),
            pl.BlockSpec((_C, _TB, _H, _W), lambda q: (0, q, 0, 0)),
        ],
        out_shape=[
            jax.ShapeDtypeStruct((_C, _NSLOW, _H, _W), jnp.float32),
            jax.ShapeDtypeStruct((_C, _T, _H, _W), jnp.float32),
        ],
    )(frames)
    return (slow, fast)
